# baseline (speedup 1.0000x reference)
import functools

import jax
import jax.numpy as jnp
from jax import lax
from jax.experimental import pallas as pl
from jax.experimental.pallas import tpu as pltpu
from jax.experimental.pallas import tpu_sc as plsc

FILTERS = 16
KSIZE = 3
GAMMA = 1.0

B, H, W, C = 4, 32, 32, 16
D = C * KSIZE * KSIZE
S2 = 2 * D
NW = 32
PIX = B * H * W

NEWTON_ITERS = 12
CAP = D + 16
SC_ROWS = 8
TC_ROWS = H - SC_ROWS
SCPIX = B * SC_ROWS * W
PPW = SCPIX // NW
ROWS_PER_W = PPW // W


def _sc_spike_conv(xb, wh, wmx):
    mesh = plsc.VectorSubcoreMesh(core_axis_name="c", subcore_axis_name="s")

    @functools.partial(
        pl.kernel,
        out_type=jax.ShapeDtypeStruct((SCPIX, FILTERS), jnp.float32),
        mesh=mesh,
        compiler_params=pltpu.CompilerParams(use_tc_tiling_on_sc=False),
        scratch_types=[
            pltpu.VMEM((ROWS_PER_W + 2, W + 2, C, FILTERS), jnp.float32),
            pltpu.VMEM((D, FILTERS), jnp.float32),
            pltpu.VMEM((D, FILTERS), jnp.float32),
            pltpu.VMEM((2, CAP, FILTERS), jnp.float32),
            pltpu.VMEM((2, CAP, FILTERS), jnp.float32),
            pltpu.VMEM((2, CAP, FILTERS), jnp.float32),
            pltpu.VMEM((PPW, FILTERS), jnp.float32),
        ],
    )
    def k(xb_hbm, wh_hbm, wmx_hbm, out_hbm, slab_v, wh_v, wmax_v,
          ma_v, mb_v, bnd_v, out_v):
        wid = lax.axis_index("s") * 2 + lax.axis_index("c")
        img = wid // (SC_ROWS // ROWS_PER_W)
        row0 = (wid % (SC_ROWS // ROWS_PER_W)) * ROWS_PER_W
        pltpu.sync_copy(xb_hbm.at[img, pl.ds(row0, ROWS_PER_W + 2)], slab_v)
        pltpu.sync_copy(wh_hbm, wh_v)
        pltpu.sync_copy(wmx_hbm, wmax_v)

        phi0 = jnp.full((FILTERS,), GAMMA / S2, dtype=jnp.float32)
        zero = jnp.zeros((FILTERS,), dtype=jnp.float32)
        negbig = jnp.full((FILTERS,), -3.0e38, dtype=jnp.float32)

        def pair_body(i, _):
            p = 2 * i
            r = p // W
            col = p - r * W

            bcarry = (zero,) * 12
            for dij in range(KSIZE * KSIZE):
                di, dj = dij // KSIZE, dij % KSIZE

                def build_c(c, carry, di=di, dj=dj, dij=dij):
                    acc = list(carry)
                    d = dij * C + c
                    wv = wh_v[d]
                    wm = wmax_v[d]
                    for s in range(2):
                        sma, ga2, ca2, smb, gb2, cb2 = acc[6 * s:6 * s + 6]
                        x = slab_v[r + di, col + s + dj, c]
                        ma = jnp.abs(x + wv)
                        mb = jnp.abs(x - wv)
                        ma_v[s, d] = ma
                        mb_v[s, d] = mb
                        bnd_v[s, d] = jnp.abs(x) + wm
                        s2a = phi0 - ma
                        s2b = phi0 - mb
                        acc[6 * s:6 * s + 6] = [
                            sma + ma,
                            ga2 + jnp.maximum(s2a, 0.0),
                            ca2 + jnp.where(s2a > 0.0, 1.0, 0.0),
                            smb + mb,
                            gb2 + jnp.maximum(s2b, 0.0),
                            cb2 + jnp.where(s2b > 0.0, 1.0, 0.0),
                        ]
                    return tuple(acc)

                bcarry = lax.fori_loop(0, C, build_c, bcarry, unroll=2)

            dphi0 = jnp.full((FILTERS,), D * (GAMMA / S2), dtype=jnp.float32)
            phis2 = []
            for s in range(2):
                sma, ga2, ca2, smb, gb2, cb2 = bcarry[6 * s:6 * s + 6]
                ga = dphi0 + sma + ga2
                gb = dphi0 + smb + gb2
                ca = ca2 + jnp.float32(D)
                cb = cb2 + jnp.float32(D)
                phis2.append((phi0 - (ga - GAMMA) / ca,
                              phi0 - (gb - GAMMA) / cb))
            phis2 = tuple(phis2)

            def newton_update(raw, phis2, npads):
                new = []
                for s in range(2):
                    pa, pb = phis2[s]
                    ga1, ga2, ca1, ca2, gb1, gb2, cb1, cb2 = raw[8 * s:8 * s + 8]
                    padf = lax.convert_element_type(2 * npads[s], jnp.float32)
                    ga = ga1 + ga2 - padf * jnp.maximum(pa, 0.0)
                    ca = ca1 + ca2 - padf * jnp.where(pa > 0.0, 1.0, 0.0)
                    gb = gb1 + gb2 - padf * jnp.maximum(pb, 0.0)
                    cb = cb1 + cb2 - padf * jnp.where(pb > 0.0, 1.0, 0.0)
                    ca = jnp.maximum(ca, 1.0)
                    cb = jnp.maximum(cb, 1.0)
                    new.append((pa - (ga - GAMMA) / ca,
                                pb - (gb - GAMMA) / cb))
                return tuple(new)

            def acc_unit(acc8, pa, pb, ma, mb):
                ga1, ga2, ca1, ca2, gb1, gb2, cb1, cb2 = acc8
                s1a = pa + ma
                s2a = pa - ma
                s1b = pb + mb
                s2b = pb - mb
                return [ga1 + jnp.maximum(s1a, 0.0),
                        ga2 + jnp.maximum(s2a, 0.0),
                        ca1 + jnp.where(s1a > 0.0, 1.0, 0.0),
                        ca2 + jnp.where(s2a > 0.0, 1.0, 0.0),
                        gb1 + jnp.maximum(s1b, 0.0),
                        gb2 + jnp.maximum(s2b, 0.0),
                        cb1 + jnp.where(s1b > 0.0, 1.0, 0.0),
                        cb2 + jnp.where(s2b > 0.0, 1.0, 0.0)]

            def newton_pair2(phis2, n8, npads):
                def blk(t, carry):
                    acc = list(carry)
                    base = t * 2
                    for j in range(2):
                        for s in range(2):
                            pa, pb = phis2[s]
                            acc[8 * s:8 * s + 8] = acc_unit(
                                acc[8 * s:8 * s + 8], pa, pb,
                                ma_v[s, base + j], mb_v[s, base + j])
                    return tuple(acc)

                raw = lax.fori_loop(0, n8 // 2, blk, (zero,) * 16)
                return newton_update(raw, phis2, npads)

            def newton_compact2(phis2, nd, npads):
                thrs = []
                for s in range(2):
                    pa, pb = phis2[s]
                    mv = jnp.maximum(pa, pb)
                    mx = mv[0]
                    for i2 in range(1, FILTERS):
                        mx = jnp.maximum(mx, mv[i2])
                    thrs.append(-mx)

                def blk(t, carry):
                    acc = list(carry[:16])
                    ns = list(carry[16:])
                    base = t * 2
                    for j in range(2):
                        for s in range(2):
                            pa, pb = phis2[s]
                            d = base + j
                            ma = ma_v[s, d]
                            mb = mb_v[s, d]
                            bv = bnd_v[s, d]
                            ma_v[s, ns[s]] = ma
                            mb_v[s, ns[s]] = mb
                            bnd_v[s, ns[s]] = bv
                            acc[8 * s:8 * s + 8] = acc_unit(
                                acc[8 * s:8 * s + 8], pa, pb, ma, mb)
                            ns[s] = ns[s] + jnp.where(bv[0] > thrs[s], 1, 0)
                    return tuple(acc) + tuple(ns)

                out = lax.fori_loop(0, nd // 2, blk, (zero,) * 16 + (0, 0))
                raw, (n0, n1) = out[:16], out[16:]
                n8 = jnp.maximum(jnp.bitwise_and(n0 + 7, -8),
                                 jnp.bitwise_and(n1 + 7, -8))

                def pad_s(s, n):
                    def w(d2, _):
                        ma_v[s, d2] = zero
                        mb_v[s, d2] = zero
                        bnd_v[s, d2] = negbig
                        return 0
                    lax.fori_loop(n, n8, w, 0)

                pad_s(0, n0)
                pad_s(1, n1)
                phis2 = newton_update(raw, phis2, npads)
                return phis2, n8, (n8 - n0, n8 - n1)

            phis2, n8, npads = newton_compact2(phis2, D, (0, 0))
            phis2 = lax.fori_loop(
                0, 2, lambda _, q: newton_pair2(q, n8, npads), phis2)
            phis2, n8b, npads2 = newton_compact2(phis2, n8, npads)
            phis2 = lax.fori_loop(
                0, NEWTON_ITERS - 5,
                lambda _, q: newton_pair2(q, n8b, npads2), phis2)

            for s in range(2):
                pa, pb = phis2[s]
                out_v[p + s] = jnp.maximum(pa - pb, 0.0)
            return 0

        lax.fori_loop(0, PPW // 2, pair_body, 0)
        pltpu.sync_copy(out_v, out_hbm.at[pl.ds(wid * PPW, PPW)])

    return k(xb, wh, wmx)


def _tc_spike_conv(xT, wh):
    P = xT.shape[1]
    nblk = P // 128

    def body(x_ref, wh_ref, o_ref, ma_ref, mb_ref):
        x = x_ref[...]
        whv = wh_ref[...]
        ma_ref[...] = jnp.abs(x[:, None, :] + whv[:, :, None])
        mb_ref[...] = jnp.abs(x[:, None, :] - whv[:, :, None])
        phi0 = jnp.full((FILTERS, 128), GAMMA / S2, dtype=jnp.float32)

        def one_pass(_, phis):
            pa, pb = phis

            def dl(d, carry):
                ga, ca, gb, cb = carry
                ma = ma_ref[d]
                mb = mb_ref[d]
                s1a = pa + ma
                s2a = pa - ma
                s1b = pb + mb
                s2b = pb - mb
                ga = ga + jnp.maximum(s1a, 0.0) + jnp.maximum(s2a, 0.0)
                ca = ca + jnp.where(s1a > 0.0, 1.0, 0.0) \
                        + jnp.where(s2a > 0.0, 1.0, 0.0)
                gb = gb + jnp.maximum(s1b, 0.0) + jnp.maximum(s2b, 0.0)
                cb = cb + jnp.where(s1b > 0.0, 1.0, 0.0) \
                        + jnp.where(s2b > 0.0, 1.0, 0.0)
                return ga, ca, gb, cb

            z = jnp.zeros((FILTERS, 128), dtype=jnp.float32)
            ga, ca, gb, cb = lax.fori_loop(0, D, dl, (z, z, z, z), unroll=2)
            pa = pa - (ga - GAMMA) / jnp.maximum(ca, 1.0)
            pb = pb - (gb - GAMMA) / jnp.maximum(cb, 1.0)
            return pa, pb

        pa, pb = lax.fori_loop(0, NEWTON_ITERS, one_pass, (phi0, phi0))
        o_ref[...] = jnp.maximum(pa - pb, 0.0)

    return pl.pallas_call(
        body,
        grid=(nblk,),
        in_specs=[
            pl.BlockSpec((D, 128), lambda i: (0, i)),
            pl.BlockSpec((D, FILTERS), lambda i: (0, 0)),
        ],
        out_specs=pl.BlockSpec((FILTERS, 128), lambda i: (0, i)),
        out_shape=jax.ShapeDtypeStruct((FILTERS, P), jnp.float32),
        scratch_shapes=[
            pltpu.VMEM((D, FILTERS, 128), jnp.float32),
            pltpu.VMEM((D, FILTERS, 128), jnp.float32),
        ],
    )(xT, wh)


def kernel(inputs, kernel):
    xpad = jnp.pad(inputs, ((0, 0), (1, 1), (1, 1), (0, 0)))
    xb = jnp.broadcast_to(xpad[..., None], xpad.shape + (FILTERS,))
    wh = kernel * 0.5
    wmx = jnp.broadcast_to(
        jnp.max(jnp.abs(wh), axis=1, keepdims=True), (D, FILTERS))
    sc_out = _sc_spike_conv(xb, wh, wmx)
    sc_part = sc_out.reshape(B, SC_ROWS, W, FILTERS)
    pats = [xpad[:, SC_ROWS + di:SC_ROWS + di + TC_ROWS, dj:dj + W, :]
            for di in range(KSIZE) for dj in range(KSIZE)]
    patches = jnp.concatenate(pats, axis=-1)
    xT = patches.reshape(B * TC_ROWS * W, D).T
    tc_out = _tc_spike_conv(xT, wh)
    tc_part = tc_out.T.reshape(B, TC_ROWS, W, FILTERS)
    return jnp.concatenate([sc_part, tc_part], axis=1)


# speedup vs baseline: 1.4052x; 1.4052x over previous
import functools

import jax
import jax.numpy as jnp
from jax import lax
from jax.experimental import pallas as pl
from jax.experimental.pallas import tpu as pltpu
from jax.experimental.pallas import tpu_sc as plsc

FILTERS = 16
KSIZE = 3
GAMMA = 1.0

B, H, W, C = 4, 32, 32, 16
D = C * KSIZE * KSIZE
S2 = 2 * D
NW = 32
PIX = B * H * W

NEWTON_ITERS = 12
CAP = D + 16
SC_ROWS = 16
TC_ROWS = H - SC_ROWS
SCPIX = B * SC_ROWS * W
PPW = SCPIX // NW
ROWS_PER_W = PPW // W


def _sc_spike_conv(xb, wh, wmx):
    mesh = plsc.VectorSubcoreMesh(core_axis_name="c", subcore_axis_name="s")

    @functools.partial(
        pl.kernel,
        out_type=jax.ShapeDtypeStruct((SCPIX, FILTERS), jnp.float32),
        mesh=mesh,
        compiler_params=pltpu.CompilerParams(use_tc_tiling_on_sc=False),
        scratch_types=[
            pltpu.VMEM((ROWS_PER_W + 2, W + 2, C, FILTERS), jnp.float32),
            pltpu.VMEM((D, FILTERS), jnp.float32),
            pltpu.VMEM((D, FILTERS), jnp.float32),
            pltpu.VMEM((2, CAP, FILTERS), jnp.float32),
            pltpu.VMEM((2, CAP, FILTERS), jnp.float32),
            pltpu.VMEM((2, CAP, FILTERS), jnp.float32),
            pltpu.VMEM((PPW, FILTERS), jnp.float32),
        ],
    )
    def k(xb_hbm, wh_hbm, wmx_hbm, out_hbm, slab_v, wh_v, wmax_v,
          ma_v, mb_v, bnd_v, out_v):
        wid = lax.axis_index("s") * 2 + lax.axis_index("c")
        img = wid // (SC_ROWS // ROWS_PER_W)
        row0 = (wid % (SC_ROWS // ROWS_PER_W)) * ROWS_PER_W
        pltpu.sync_copy(xb_hbm.at[img, pl.ds(row0, ROWS_PER_W + 2)], slab_v)
        pltpu.sync_copy(wh_hbm, wh_v)
        pltpu.sync_copy(wmx_hbm, wmax_v)

        phi0 = jnp.full((FILTERS,), GAMMA / S2, dtype=jnp.float32)
        zero = jnp.zeros((FILTERS,), dtype=jnp.float32)
        negbig = jnp.full((FILTERS,), -3.0e38, dtype=jnp.float32)

        def pair_body(i, _):
            p = 2 * i
            r = p // W
            col = p - r * W

            bcarry = (zero,) * 12
            for dij in range(KSIZE * KSIZE):
                di, dj = dij // KSIZE, dij % KSIZE

                def build_c(c, carry, di=di, dj=dj, dij=dij):
                    acc = list(carry)
                    d = dij * C + c
                    wv = wh_v[d]
                    wm = wmax_v[d]
                    for s in range(2):
                        sma, ga2, ca2, smb, gb2, cb2 = acc[6 * s:6 * s + 6]
                        x = slab_v[r + di, col + s + dj, c]
                        ma = jnp.abs(x + wv)
                        mb = jnp.abs(x - wv)
                        ma_v[s, d] = ma
                        mb_v[s, d] = mb
                        bnd_v[s, d] = jnp.abs(x) + wm
                        s2a = phi0 - ma
                        s2b = phi0 - mb
                        acc[6 * s:6 * s + 6] = [
                            sma + ma,
                            ga2 + jnp.maximum(s2a, 0.0),
                            ca2 + jnp.where(s2a > 0.0, 1.0, 0.0),
                            smb + mb,
                            gb2 + jnp.maximum(s2b, 0.0),
                            cb2 + jnp.where(s2b > 0.0, 1.0, 0.0),
                        ]
                    return tuple(acc)

                bcarry = lax.fori_loop(0, C, build_c, bcarry, unroll=2)

            dphi0 = jnp.full((FILTERS,), D * (GAMMA / S2), dtype=jnp.float32)
            phis2 = []
            for s in range(2):
                sma, ga2, ca2, smb, gb2, cb2 = bcarry[6 * s:6 * s + 6]
                ga = dphi0 + sma + ga2
                gb = dphi0 + smb + gb2
                ca = ca2 + jnp.float32(D)
                cb = cb2 + jnp.float32(D)
                phis2.append((phi0 - (ga - GAMMA) / ca,
                              phi0 - (gb - GAMMA) / cb))
            phis2 = tuple(phis2)

            def newton_update(raw, phis2, npads):
                new = []
                for s in range(2):
                    pa, pb = phis2[s]
                    ga1, ga2, ca1, ca2, gb1, gb2, cb1, cb2 = raw[8 * s:8 * s + 8]
                    padf = lax.convert_element_type(2 * npads[s], jnp.float32)
                    ga = ga1 + ga2 - padf * jnp.maximum(pa, 0.0)
                    ca = ca1 + ca2 - padf * jnp.where(pa > 0.0, 1.0, 0.0)
                    gb = gb1 + gb2 - padf * jnp.maximum(pb, 0.0)
                    cb = cb1 + cb2 - padf * jnp.where(pb > 0.0, 1.0, 0.0)
                    ca = jnp.maximum(ca, 1.0)
                    cb = jnp.maximum(cb, 1.0)
                    new.append((pa - (ga - GAMMA) / ca,
                                pb - (gb - GAMMA) / cb))
                return tuple(new)

            def acc_unit(acc8, pa, pb, ma, mb):
                ga1, ga2, ca1, ca2, gb1, gb2, cb1, cb2 = acc8
                s1a = pa + ma
                s2a = pa - ma
                s1b = pb + mb
                s2b = pb - mb
                return [ga1 + jnp.maximum(s1a, 0.0),
                        ga2 + jnp.maximum(s2a, 0.0),
                        ca1 + jnp.where(s1a > 0.0, 1.0, 0.0),
                        ca2 + jnp.where(s2a > 0.0, 1.0, 0.0),
                        gb1 + jnp.maximum(s1b, 0.0),
                        gb2 + jnp.maximum(s2b, 0.0),
                        cb1 + jnp.where(s1b > 0.0, 1.0, 0.0),
                        cb2 + jnp.where(s2b > 0.0, 1.0, 0.0)]

            def newton_pair2(phis2, n8, npads):
                def blk(t, carry):
                    acc = list(carry)
                    base = t * 2
                    for j in range(2):
                        for s in range(2):
                            pa, pb = phis2[s]
                            acc[8 * s:8 * s + 8] = acc_unit(
                                acc[8 * s:8 * s + 8], pa, pb,
                                ma_v[s, base + j], mb_v[s, base + j])
                    return tuple(acc)

                raw = lax.fori_loop(0, n8 // 2, blk, (zero,) * 16)
                return newton_update(raw, phis2, npads)

            def newton_compact2(phis2, nd, npads):
                thrs = []
                for s in range(2):
                    pa, pb = phis2[s]
                    mv = jnp.maximum(pa, pb)
                    mx = mv[0]
                    for i2 in range(1, FILTERS):
                        mx = jnp.maximum(mx, mv[i2])
                    thrs.append(-mx)

                def blk(t, carry):
                    acc = list(carry[:16])
                    ns = list(carry[16:])
                    base = t * 2
                    for j in range(2):
                        for s in range(2):
                            pa, pb = phis2[s]
                            d = base + j
                            ma = ma_v[s, d]
                            mb = mb_v[s, d]
                            bv = bnd_v[s, d]
                            ma_v[s, ns[s]] = ma
                            mb_v[s, ns[s]] = mb
                            bnd_v[s, ns[s]] = bv
                            acc[8 * s:8 * s + 8] = acc_unit(
                                acc[8 * s:8 * s + 8], pa, pb, ma, mb)
                            ns[s] = ns[s] + jnp.where(bv[0] > thrs[s], 1, 0)
                    return tuple(acc) + tuple(ns)

                out = lax.fori_loop(0, nd // 2, blk, (zero,) * 16 + (0, 0))
                raw, (n0, n1) = out[:16], out[16:]
                n8 = jnp.maximum(jnp.bitwise_and(n0 + 7, -8),
                                 jnp.bitwise_and(n1 + 7, -8))

                def pad_s(s, n):
                    def w(d2, _):
                        ma_v[s, d2] = zero
                        mb_v[s, d2] = zero
                        bnd_v[s, d2] = negbig
                        return 0
                    lax.fori_loop(n, n8, w, 0)

                pad_s(0, n0)
                pad_s(1, n1)
                phis2 = newton_update(raw, phis2, npads)
                return phis2, n8, (n8 - n0, n8 - n1)

            phis2, n8, npads = newton_compact2(phis2, D, (0, 0))
            phis2 = lax.fori_loop(
                0, 2, lambda _, q: newton_pair2(q, n8, npads), phis2)
            phis2, n8b, npads2 = newton_compact2(phis2, n8, npads)
            phis2 = lax.fori_loop(
                0, NEWTON_ITERS - 5,
                lambda _, q: newton_pair2(q, n8b, npads2), phis2)

            for s in range(2):
                pa, pb = phis2[s]
                out_v[p + s] = jnp.maximum(pa - pb, 0.0)
            return 0

        lax.fori_loop(0, PPW // 2, pair_body, 0)
        pltpu.sync_copy(out_v, out_hbm.at[pl.ds(wid * PPW, PPW)])

    return k(xb, wh, wmx)


def _tc_spike_conv(xT, wh):
    P = xT.shape[1]
    nblk = P // 128

    def body(x_ref, wh_ref, o_ref, ma_ref, mb_ref):
        x = x_ref[...]
        whv = wh_ref[...]
        ma_ref[...] = jnp.abs(x[:, None, :] + whv[:, :, None])
        mb_ref[...] = jnp.abs(x[:, None, :] - whv[:, :, None])
        phi0 = jnp.full((FILTERS, 128), GAMMA / S2, dtype=jnp.float32)

        def one_pass(_, phis):
            pa, pb = phis

            def dl(d, carry):
                ga, ca, gb, cb = carry
                ma = ma_ref[d]
                mb = mb_ref[d]
                s1a = pa + ma
                s2a = pa - ma
                s1b = pb + mb
                s2b = pb - mb
                ga = ga + jnp.maximum(s1a, 0.0) + jnp.maximum(s2a, 0.0)
                ca = ca + jnp.where(s1a > 0.0, 1.0, 0.0) \
                        + jnp.where(s2a > 0.0, 1.0, 0.0)
                gb = gb + jnp.maximum(s1b, 0.0) + jnp.maximum(s2b, 0.0)
                cb = cb + jnp.where(s1b > 0.0, 1.0, 0.0) \
                        + jnp.where(s2b > 0.0, 1.0, 0.0)
                return ga, ca, gb, cb

            z = jnp.zeros((FILTERS, 128), dtype=jnp.float32)
            ga, ca, gb, cb = lax.fori_loop(0, D, dl, (z, z, z, z), unroll=2)
            pa = pa - (ga - GAMMA) / jnp.maximum(ca, 1.0)
            pb = pb - (gb - GAMMA) / jnp.maximum(cb, 1.0)
            return pa, pb

        pa, pb = lax.fori_loop(0, NEWTON_ITERS, one_pass, (phi0, phi0))
        o_ref[...] = jnp.maximum(pa - pb, 0.0)

    return pl.pallas_call(
        body,
        grid=(nblk,),
        in_specs=[
            pl.BlockSpec((D, 128), lambda i: (0, i)),
            pl.BlockSpec((D, FILTERS), lambda i: (0, 0)),
        ],
        out_specs=pl.BlockSpec((FILTERS, 128), lambda i: (0, i)),
        out_shape=jax.ShapeDtypeStruct((FILTERS, P), jnp.float32),
        scratch_shapes=[
            pltpu.VMEM((D, FILTERS, 128), jnp.float32),
            pltpu.VMEM((D, FILTERS, 128), jnp.float32),
        ],
    )(xT, wh)


def kernel(inputs, kernel):
    xpad = jnp.pad(inputs, ((0, 0), (1, 1), (1, 1), (0, 0)))
    xb = jnp.broadcast_to(xpad[..., None], xpad.shape + (FILTERS,))
    wh = kernel * 0.5
    wmx = jnp.broadcast_to(
        jnp.max(jnp.abs(wh), axis=1, keepdims=True), (D, FILTERS))
    sc_out = _sc_spike_conv(xb, wh, wmx)
    sc_part = sc_out.reshape(B, SC_ROWS, W, FILTERS)
    pats = [xpad[:, SC_ROWS + di:SC_ROWS + di + TC_ROWS, dj:dj + W, :]
            for di in range(KSIZE) for dj in range(KSIZE)]
    patches = jnp.concatenate(pats, axis=-1)
    xT = patches.reshape(B * TC_ROWS * W, D).T
    tc_out = _tc_spike_conv(xT, wh)
    tc_part = tc_out.T.reshape(B, TC_ROWS, W, FILTERS)
    return jnp.concatenate([sc_part, tc_part], axis=1)
